# baseline (device time: 220685 ns/iter reference)
import jax
import jax.numpy as jnp
from jax import lax
from jax.experimental import pallas as pl
from jax.experimental.pallas import tpu as pltpu

NB = 4
NBC = 8
NA = 4


def kernel(A, B):
    M, K = A.shape
    K2, N = B.shape
    assert K == K2
    HALF = N // 2
    CB = HALF // NB
    KC = K // NA

    A = A.astype(jnp.bfloat16)
    B = B.astype(jnp.bfloat16)

    def body(a_hbm, b_hbm, out_ref, a_vmem, b_vmem, out_vmem, a_r, b_r,
             a_in, b_in, a_send, a_recv, b_send, b_recv, f_send, f_recv,
             cp_sems):
        my_x = lax.axis_index("x")
        my_y = lax.axis_index("y")
        py = (my_x, 1 - my_y)
        px = (1 - my_x, my_y)

        barrier = pltpu.get_barrier_semaphore()
        for nbr in (py, px):
            pl.semaphore_signal(
                barrier, inc=1, device_id=nbr,
                device_id_type=pl.DeviceIdType.MESH,
            )
        pl.semaphore_wait(barrier, 2)

        dbase = my_x * HALF

        def bcol(g):
            return pl.ds(lax.rem(dbase + g * CB, N), CB)

        a_copies = []
        for c in range(NA):
            kc = pl.ds(c * KC, KC)
            cp = pltpu.make_async_copy(
                a_hbm.at[:, kc], a_vmem.at[:, kc], a_in.at[c])
            cp.start()
            a_copies.append(cp)
        b_copies = []
        for g in range(NBC):
            cp = pltpu.make_async_copy(
                b_hbm.at[:, bcol(g)], b_vmem.at[:, bcol(g)], b_in.at[g])
            cp.start()
            b_copies.append(cp)

        a_rdmas = []
        for c in range(NA):
            kc = pl.ds(c * KC, KC)
            a_copies[c].wait()
            r = pltpu.make_async_remote_copy(
                src_ref=a_vmem.at[:, kc],
                dst_ref=a_r.at[:, kc],
                send_sem=a_send.at[c],
                recv_sem=a_recv.at[c],
                device_id=py,
                device_id_type=pl.DeviceIdType.MESH,
            )
            r.start()
            a_rdmas.append(r)

        b_rdmas = []
        for g in range(NB):
            b_copies[g].wait()
            r = pltpu.make_async_remote_copy(
                src_ref=b_vmem.at[:, bcol(g)],
                dst_ref=b_r.at[:, bcol(g)],
                send_sem=b_send.at[g],
                recv_sem=b_recv.at[g],
                device_id=py,
                device_id_type=pl.DeviceIdType.MESH,
            )
            r.start()
            b_rdmas.append(r)

        for g in range(NBC):
            if g >= NB:
                b_copies[g].wait()
            out_vmem[:, bcol(g)] = jnp.dot(
                a_vmem[:, :], b_vmem[:, bcol(g)],
                preferred_element_type=jnp.float32,
            ).astype(jnp.bfloat16)

        for c in range(NA):
            a_rdmas[c].wait_recv()

        f_rdmas = []
        copies = []
        for g in range(NB):
            cols = bcol(g)
            b_rdmas[g].wait_recv()
            f = pltpu.make_async_remote_copy(
                src_ref=b_r.at[:, cols],
                dst_ref=b_r.at[:, cols],
                send_sem=f_send.at[g],
                recv_sem=f_recv.at[g],
                device_id=px,
                device_id_type=pl.DeviceIdType.MESH,
            )
            f.start()
            f_rdmas.append(f)
            out_vmem[:, cols] = (
                out_vmem[:, cols].astype(jnp.float32)
                + jnp.dot(a_r[:, :], b_r[:, cols],
                          preferred_element_type=jnp.float32)
            ).astype(jnp.bfloat16)
            cp = pltpu.make_async_copy(
                out_vmem.at[:, cols], out_ref.at[:, cols], cp_sems.at[g])
            cp.start()
            copies.append(cp)

        for g in range(NB):
            cols = bcol(NB + g)
            rin = pltpu.make_async_remote_copy(
                src_ref=b_r.at[:, cols],
                dst_ref=b_r.at[:, cols],
                send_sem=f_send.at[g],
                recv_sem=f_recv.at[g],
                device_id=px,
                device_id_type=pl.DeviceIdType.MESH,
            )
            rin.wait_recv()
            out_vmem[:, cols] = (
                out_vmem[:, cols].astype(jnp.float32)
                + jnp.dot(a_r[:, :], b_r[:, cols],
                          preferred_element_type=jnp.float32)
            ).astype(jnp.bfloat16)
            cp = pltpu.make_async_copy(
                out_vmem.at[:, cols], out_ref.at[:, cols],
                cp_sems.at[NB + g])
            cp.start()
            copies.append(cp)

        for c in range(NA):
            a_rdmas[c].wait_send()
        for g in range(NB):
            b_rdmas[g].wait_send()
            f_rdmas[g].wait_send()
        for cp in copies:
            cp.wait()

    return pl.pallas_call(
        body,
        out_shape=jax.ShapeDtypeStruct((M, N), jnp.bfloat16),
        in_specs=[
            pl.BlockSpec(memory_space=pl.ANY),
            pl.BlockSpec(memory_space=pl.ANY),
        ],
        out_specs=pl.BlockSpec(memory_space=pl.ANY),
        scratch_shapes=[
            pltpu.VMEM((M, K), jnp.bfloat16),
            pltpu.VMEM((K, N), jnp.bfloat16),
            pltpu.VMEM((M, N), jnp.bfloat16),
            pltpu.VMEM((M, K), jnp.bfloat16),
            pltpu.VMEM((K, N), jnp.bfloat16),
            pltpu.SemaphoreType.DMA((NA,)),
            pltpu.SemaphoreType.DMA((NBC,)),
            pltpu.SemaphoreType.DMA((NA,)),
            pltpu.SemaphoreType.DMA((NA,)),
            pltpu.SemaphoreType.DMA((NB,)),
            pltpu.SemaphoreType.DMA((NB,)),
            pltpu.SemaphoreType.DMA((NB,)),
            pltpu.SemaphoreType.DMA((NB,)),
            pltpu.SemaphoreType.DMA((2 * NB,)),
        ],
        compiler_params=pltpu.CompilerParams(
            collective_id=0,
            vmem_limit_bytes=62 * 1024 * 1024,
        ),
    )(A, B)
